# Initial kernel scaffold; baseline (speedup 1.0000x reference)
import jax
import jax.numpy as jnp
from jax.experimental import pallas as pl


def kernel(x, edge_index, edge_weight, W0, b0, W1, b1, Wp, bp, g0, be0, g1, be1):
    raise NotImplementedError("write your pallas kernel here")


# speedup vs baseline: 17.1965x; 17.1965x over previous
import functools

import jax
import jax.numpy as jnp
from jax import lax
from jax.experimental import pallas as pl
from jax.experimental.pallas import tpu as pltpu
from jax.experimental.pallas import tpu_sc as plsc

N = 10000
E = 320000
D_IN = 128
D_H = 64

NC = 2
NS = 16
NW = NC * NS
L = 16

CH = 128
NCH = -(-(E // NW) // CH)
EPW = NCH * CH
NP = 10240
NSTR = NP // NS

_mesh = plsc.VectorSubcoreMesh(
    core_axis_name="c", subcore_axis_name="s", num_cores=NC, num_subcores=NS)


def _deg_body(dst_hbm, w_hbm, out_hbm, dstv, wv, acc):
    c = lax.axis_index("c")
    s = lax.axis_index("s")
    wid = c * NS + s
    pltpu.sync_copy(dst_hbm.at[wid], dstv)
    pltpu.sync_copy(w_hbm.at[wid], wv)
    z16 = jnp.zeros((L,), jnp.float32)

    def zb(i, carry):
        acc[pl.ds(i * L, L)] = z16
        return carry

    lax.fori_loop(0, N // L, zb, 0)

    def eb(j, carry):
        for g in range(CH // L):
            d16 = dstv[j, pl.ds(g * L, L)]
            w16 = wv[j, pl.ds(g * L, L)]
            plsc.addupdate_scatter(acc, [d16], w16)
        return carry

    lax.fori_loop(0, NCH, eb, 0)
    pltpu.sync_copy(acc, out_hbm.at[wid])


_sc_params = pltpu.CompilerParams(
    needs_layout_passes=False, use_tc_tiling_on_sc=False)

_deg_kernel = functools.partial(
    pl.kernel,
    out_type=jax.ShapeDtypeStruct((NW, N), jnp.float32),
    mesh=_mesh,
    compiler_params=_sc_params,
    scratch_types=[
        pltpu.VMEM((NCH, CH), jnp.int32),
        pltpu.VMEM((NCH, CH), jnp.float32),
        pltpu.VMEM((N,), jnp.float32),
    ],
)(_deg_body)


def _agg_body(ht_hbm, src_hbm, dst_hbm, w_hbm, out_hbm,
              srcv, dstv, wv, rows, stage, acc, sem):
    c = lax.axis_index("c")
    s = lax.axis_index("s")
    wid = c * NS + s
    pltpu.sync_copy(src_hbm.at[wid], srcv)
    pltpu.sync_copy(dst_hbm.at[wid], dstv)
    pltpu.sync_copy(w_hbm.at[wid], wv)

    z16 = jnp.zeros((L,), jnp.float32)

    def zb(i, carry):
        for g in range(D_H // L):
            stage[i, pl.ds(g * L, L)] = z16
        return carry

    lax.fori_loop(0, NSTR, zb, 0)
    pltpu.sync_copy(stage, acc.at[pl.ds(s * NSTR, NSTR)])
    plsc.subcore_barrier()

    def eb(j, carry):
        pltpu.async_copy(ht_hbm.at[srcv.at[j]], rows, sem).wait()
        for g in range(CH // L):
            w16 = wv[j, pl.ds(g * L, L)]
            for e in range(L):
                we = jnp.take_along_axis(
                    w16, jnp.full((L,), e, jnp.int32), axis=0,
                    mode=lax.GatherScatterMode.PROMISE_IN_BOUNDS)
                r = g * L + e
                for k in range(D_H // L):
                    rows[r, pl.ds(k * L, L)] = rows[r, pl.ds(k * L, L)] * we
        pltpu.sync_copy(rows, acc.at[dstv.at[j]], add=True)
        return carry

    lax.fori_loop(0, NCH, eb, 0)
    plsc.subcore_barrier()
    pltpu.sync_copy(acc.at[pl.ds(s * NSTR, NSTR)], stage)
    pltpu.sync_copy(stage, out_hbm.at[wid])


_agg_kernel = functools.partial(
    pl.kernel,
    out_type=jax.ShapeDtypeStruct((NW, NSTR, D_H), jnp.float32),
    mesh=_mesh,
    compiler_params=_sc_params,
    scratch_types=[
        pltpu.VMEM((NCH, CH), jnp.int32),
        pltpu.VMEM((NCH, CH), jnp.int32),
        pltpu.VMEM((NCH, CH), jnp.float32),
        pltpu.VMEM((CH, D_H), jnp.float32),
        pltpu.VMEM((NSTR, D_H), jnp.float32),
        pltpu.VMEM_SHARED((NP, D_H), jnp.float32),
        pltpu.SemaphoreType.DMA,
    ],
)(_agg_body)


def _p0_body(x_ref, wp_ref, bp_ref, w0_ref, degp_ref,
             dis_ref, res_ref, ht0_ref):
    deg = 1.0 + jnp.sum(degp_ref[...], axis=0)
    dis = lax.rsqrt(deg)
    dis_ref[...] = dis
    x = x_ref[...]
    res_ref[...] = (
        jnp.dot(x, wp_ref[...], preferred_element_type=jnp.float32)
        + bp_ref[...][None, :])
    ht0_ref[...] = (
        jnp.dot(x, w0_ref[...], preferred_element_type=jnp.float32)
        * dis[:, None])


def _p0(x, Wp, bp, W0, degp):
    return pl.pallas_call(
        _p0_body,
        out_shape=(
            jax.ShapeDtypeStruct((N,), jnp.float32),
            jax.ShapeDtypeStruct((N, D_H), jnp.float32),
            jax.ShapeDtypeStruct((N, D_H), jnp.float32),
        ),
    )(x, Wp, bp, W0, degp)


def _post_body(has_proj, aggp_ref, ht_ref, dis_ref, b_ref, g_ref, be_ref,
               skip_ref, *rest):
    if has_proj:
        wn_ref, h_ref, htn_ref = rest
    else:
        (h_ref,) = rest
    dis = dis_ref[...]
    agg = aggp_ref[0] + aggp_ref[1] + ht_ref[...]
    pre = agg * dis[:, None] + b_ref[...][None, :]
    m = jnp.mean(pre, axis=0)
    v = jnp.mean((pre - m[None, :]) ** 2, axis=0)
    hb = (pre - m[None, :]) * lax.rsqrt(v + 1e-5)[None, :]
    hb = hb * g_ref[...][None, :] + be_ref[...][None, :]
    h = jnp.maximum(hb, 0.0) + skip_ref[...]
    h_ref[...] = h
    if has_proj:
        htn_ref[...] = (
            jnp.dot(h, wn_ref[...], preferred_element_type=jnp.float32)
            * dis[:, None])


def _post(aggp, ht, dis, b, g, be, skip, Wn=None):
    if Wn is not None:
        return pl.pallas_call(
            functools.partial(_post_body, True),
            out_shape=(
                jax.ShapeDtypeStruct((N, D_H), jnp.float32),
                jax.ShapeDtypeStruct((N, D_H), jnp.float32),
            ),
        )(aggp, ht, dis, b, g, be, skip, Wn)
    return pl.pallas_call(
        functools.partial(_post_body, False),
        out_shape=jax.ShapeDtypeStruct((N, D_H), jnp.float32),
    )(aggp, ht, dis, b, g, be, skip)


def kernel(x, edge_index, edge_weight, W0, b0, W1, b1, Wp, bp,
           g0, be0, g1, be1):
    src = edge_index[0]
    dst = edge_index[1]
    pad = NW * EPW - E
    srcp = jnp.concatenate(
        [src, jnp.zeros((pad,), jnp.int32)]).reshape(NW, NCH, CH)
    dstp = jnp.concatenate(
        [dst, jnp.zeros((pad,), jnp.int32)]).reshape(NW, NCH, CH)
    wp_e = jnp.concatenate(
        [edge_weight, jnp.zeros((pad,), jnp.float32)]).reshape(NW, NCH, CH)

    degp = _deg_kernel(dstp, wp_e)
    dis, resid, ht0 = _p0(x, Wp, bp, W0, degp)

    agg0 = _agg_kernel(ht0, srcp, dstp, wp_e).reshape(NC, NP, D_H)[:, :N]
    h, ht1 = _post(agg0, ht0, dis, b0, g0, be0, resid, W1)

    agg1 = _agg_kernel(ht1, srcp, dstp, wp_e).reshape(NC, NP, D_H)[:, :N]
    out = _post(agg1, ht1, dis, b1, g1, be1, h)
    return out
